# initial kernel scaffold (unmeasured)
import jax
import jax.numpy as jnp
from jax import lax
from jax.experimental import pallas as pl
from jax.experimental.pallas import tpu as pltpu

N_DEV = 4
M, N = 8192, 4096
TILE_M = 512
N_TILES = M // TILE_M


def kernel(x, w_mat):
    p = jnp.dot(x, w_mat, preferred_element_type=jnp.float32)

    def body(p_ref, out_ref, comm_ref, send_sems, recv_sems, bar_sem):
        my = lax.axis_index("i")
        left = (my - 1) % N_DEV
        right = (my + 1) % N_DEV

        out_ref[:, :] = p_ref[:, :]
        comm_ref[0, :, :] = p_ref[:, :]

        for h in range(N_DEV - 1):
            s_slot = h % 2
            r_slot = (h + 1) % 2
            rdma = pltpu.make_async_remote_copy(
                src_ref=comm_ref.at[s_slot],
                dst_ref=comm_ref.at[r_slot],
                send_sem=send_sems.at[s_slot],
                recv_sem=recv_sems.at[r_slot],
                device_id=(right,),
                device_id_type=pl.DeviceIdType.MESH,
            )
            rdma.start()
            rdma.wait()
            out_ref[:, :] = out_ref[:, :] + comm_ref[r_slot, :, :]

            for nbr in (left, right):
                pl.semaphore_signal(
                    bar_sem,
                    inc=1,
                    device_id=(nbr,),
                    device_id_type=pl.DeviceIdType.MESH,
                )
            pl.semaphore_wait(bar_sem, 2)

        y = out_ref[:, :]
        out_ref[:, :] = y / (1.0 + jnp.exp(-y))

    return pl.pallas_call(
        body,
        grid=(N_TILES,),
        out_shape=jax.ShapeDtypeStruct((M, N), jnp.float32),
        in_specs=[pl.BlockSpec((TILE_M, N), lambda i: (i, 0))],
        out_specs=pl.BlockSpec((TILE_M, N), lambda i: (i, 0)),
        scratch_shapes=[
            pltpu.VMEM((2, TILE_M, N), jnp.float32),
            pltpu.SemaphoreType.DMA((2,)),
            pltpu.SemaphoreType.DMA((2,)),
            pltpu.SemaphoreType.REGULAR,
        ],
        compiler_params=pltpu.CompilerParams(
            dimension_semantics=("arbitrary",),
        ),
    )(p)


# baseline (device time: 4818151 ns/iter reference)
import jax
import jax.numpy as jnp
from jax import lax
from jax.experimental import pallas as pl
from jax.experimental.pallas import tpu as pltpu

N_DEV = 4
M, N = 8192, 4096
TILE_M = 512
N_TILES = M // TILE_M


def kernel(x, w_mat):
    p = jnp.dot(x, w_mat, preferred_element_type=jnp.float32)

    def body(p_ref, out_ref, comm_ref, send_sems, recv_sems, bar_sem):
        my = lax.axis_index("i")
        left = (my - 1) % N_DEV
        right = (my + 1) % N_DEV

        out_ref[:, :] = p_ref[:, :]
        comm_ref[0, :, :] = p_ref[:, :]

        for h in range(N_DEV - 1):
            s_slot = h % 2
            r_slot = (h + 1) % 2
            rdma = pltpu.make_async_remote_copy(
                src_ref=comm_ref.at[s_slot],
                dst_ref=comm_ref.at[r_slot],
                send_sem=send_sems.at[s_slot],
                recv_sem=recv_sems.at[r_slot],
                device_id=(right,),
                device_id_type=pl.DeviceIdType.MESH,
            )
            rdma.start()
            rdma.wait()
            out_ref[:, :] = out_ref[:, :] + comm_ref[r_slot, :, :]

            for nbr in (left, right):
                pl.semaphore_signal(
                    bar_sem,
                    inc=1,
                    device_id=(nbr,),
                    device_id_type=pl.DeviceIdType.MESH,
                )
            pl.semaphore_wait(bar_sem, 2)

        y = out_ref[:, :]
        out_ref[:, :] = y / (1.0 + jnp.exp(-y))

    return pl.pallas_call(
        body,
        grid=(N_TILES,),
        out_shape=jax.ShapeDtypeStruct((M, N), jnp.float32),
        in_specs=[pl.BlockSpec((TILE_M, N), lambda i: (i, 0))],
        out_specs=pl.BlockSpec((TILE_M, N), lambda i: (i, 0)),
        scratch_shapes=[
            pltpu.VMEM((2, TILE_M, N), jnp.float32),
            pltpu.SemaphoreType.DMA((2,)),
            pltpu.SemaphoreType.DMA((2,)),
            pltpu.SemaphoreType.REGULAR,
        ],
        compiler_params=pltpu.CompilerParams(
            dimension_semantics=("arbitrary",),
            vmem_limit_bytes=100 * 1024 * 1024,
        ),
    )(p)


# device time: 2715389 ns/iter; 1.7744x vs baseline; 1.7744x over previous
import jax
import jax.numpy as jnp
from jax import lax
from jax.experimental import pallas as pl
from jax.experimental.pallas import tpu as pltpu

N_DEV = 4
M, N = 8192, 4096
TILE_M = 512
N_TILES = M // TILE_M
CHUNK_N = N // N_DEV


def kernel(x, w_mat):
    p = jnp.dot(x, w_mat, preferred_element_type=jnp.float32)

    def body(p_ref, out_ref, comm_ref, send_sems, recv_sems, bar_sem):
        my = lax.axis_index("i")
        left = (my - 1) % N_DEV
        right = (my + 1) % N_DEV

        def barrier():
            for nbr in (left, right):
                pl.semaphore_signal(
                    bar_sem,
                    inc=1,
                    device_id=(nbr,),
                    device_id_type=pl.DeviceIdType.MESH,
                )
            pl.semaphore_wait(bar_sem, 2)

        def hop(ks, kr, slot, accumulate):
            rdma = pltpu.make_async_remote_copy(
                src_ref=out_ref.at[:, pl.ds(ks * CHUNK_N, CHUNK_N)],
                dst_ref=comm_ref.at[slot],
                send_sem=send_sems.at[slot],
                recv_sem=recv_sems.at[slot],
                device_id=(right,),
                device_id_type=pl.DeviceIdType.MESH,
            )
            rdma.start()
            rdma.wait()
            dst = out_ref.at[:, pl.ds(kr * CHUNK_N, CHUNK_N)]
            if accumulate:
                dst[:, :] = dst[:, :] + comm_ref[slot, :, :]
            else:
                dst[:, :] = comm_ref[slot, :, :]
            barrier()

        out_ref[:, :] = p_ref[:, :]

        for s in range(N_DEV - 1):
            hop((my - s) % N_DEV, (my - s - 1) % N_DEV, s % 2, True)

        ko = (my + 1) % N_DEV
        own = out_ref.at[:, pl.ds(ko * CHUNK_N, CHUNK_N)]
        y = own[:, :]
        own[:, :] = y / (1.0 + jnp.exp(-y))

        for s in range(N_DEV - 1):
            hop((my + 1 - s) % N_DEV, (my - s) % N_DEV, s % 2, False)

    return pl.pallas_call(
        body,
        grid=(N_TILES,),
        out_shape=jax.ShapeDtypeStruct((M, N), jnp.float32),
        in_specs=[pl.BlockSpec((TILE_M, N), lambda i: (i, 0))],
        out_specs=pl.BlockSpec((TILE_M, N), lambda i: (i, 0)),
        scratch_shapes=[
            pltpu.VMEM((2, TILE_M, CHUNK_N), jnp.float32),
            pltpu.SemaphoreType.DMA((2,)),
            pltpu.SemaphoreType.DMA((2,)),
            pltpu.SemaphoreType.REGULAR,
        ],
        compiler_params=pltpu.CompilerParams(
            dimension_semantics=("arbitrary",),
            vmem_limit_bytes=100 * 1024 * 1024,
        ),
    )(p)


# device time: 1636716 ns/iter; 2.9438x vs baseline; 1.6590x over previous
import jax
import jax.numpy as jnp
from jax import lax
from jax.experimental import pallas as pl
from jax.experimental.pallas import tpu as pltpu

N_DEV = 4
M, N = 8192, 4096
TILE_M = 512
N_TILES = M // TILE_M
CHUNK_N = N // N_DEV
HALF = CHUNK_N // 2


def kernel(x, w_mat):
    p = jnp.dot(x, w_mat, preferred_element_type=jnp.float32)

    def body(p_ref, out_ref, comm_r, comm_l, send_r, recv_r, send_l, recv_l,
             bar_sem):
        my = lax.axis_index("i")
        left = (my - 1) % N_DEV
        right = (my + 1) % N_DEV

        def barrier():
            for nbr in (left, right):
                pl.semaphore_signal(
                    bar_sem,
                    inc=1,
                    device_id=(nbr,),
                    device_id_type=pl.DeviceIdType.MESH,
                )
            pl.semaphore_wait(bar_sem, 2)

        def hop2(ks_r, kr_r, ks_l, kr_l, slot, accumulate):
            rdma_r = pltpu.make_async_remote_copy(
                src_ref=out_ref.at[:, pl.ds(ks_r * CHUNK_N, HALF)],
                dst_ref=comm_r.at[slot],
                send_sem=send_r.at[slot],
                recv_sem=recv_r.at[slot],
                device_id=(right,),
                device_id_type=pl.DeviceIdType.MESH,
            )
            rdma_l = pltpu.make_async_remote_copy(
                src_ref=out_ref.at[:, pl.ds(ks_l * CHUNK_N + HALF, HALF)],
                dst_ref=comm_l.at[slot],
                send_sem=send_l.at[slot],
                recv_sem=recv_l.at[slot],
                device_id=(left,),
                device_id_type=pl.DeviceIdType.MESH,
            )
            rdma_r.start()
            rdma_l.start()
            rdma_r.wait()
            rdma_l.wait()
            dst_r = out_ref.at[:, pl.ds(kr_r * CHUNK_N, HALF)]
            dst_l = out_ref.at[:, pl.ds(kr_l * CHUNK_N + HALF, HALF)]
            if accumulate:
                dst_r[:, :] = dst_r[:, :] + comm_r[slot, :, :]
                dst_l[:, :] = dst_l[:, :] + comm_l[slot, :, :]
            else:
                dst_r[:, :] = comm_r[slot, :, :]
                dst_l[:, :] = comm_l[slot, :, :]
            barrier()

        out_ref[:, :] = p_ref[:, :]

        for s in range(N_DEV - 1):
            hop2(
                (my - s) % N_DEV, (my - s - 1) % N_DEV,
                (my + s) % N_DEV, (my + s + 1) % N_DEV,
                s % 2, True,
            )

        ko_r = (my + 1) % N_DEV
        ko_l = (my - 1) % N_DEV
        own_r = out_ref.at[:, pl.ds(ko_r * CHUNK_N, HALF)]
        own_l = out_ref.at[:, pl.ds(ko_l * CHUNK_N + HALF, HALF)]
        y = own_r[:, :]
        own_r[:, :] = y / (1.0 + jnp.exp(-y))
        y = own_l[:, :]
        own_l[:, :] = y / (1.0 + jnp.exp(-y))

        for s in range(N_DEV - 1):
            hop2(
                (my + 1 - s) % N_DEV, (my - s) % N_DEV,
                (my - 1 + s) % N_DEV, (my + s) % N_DEV,
                s % 2, False,
            )

    return pl.pallas_call(
        body,
        grid=(N_TILES,),
        out_shape=jax.ShapeDtypeStruct((M, N), jnp.float32),
        in_specs=[pl.BlockSpec((TILE_M, N), lambda i: (i, 0))],
        out_specs=pl.BlockSpec((TILE_M, N), lambda i: (i, 0)),
        scratch_shapes=[
            pltpu.VMEM((2, TILE_M, HALF), jnp.float32),
            pltpu.VMEM((2, TILE_M, HALF), jnp.float32),
            pltpu.SemaphoreType.DMA((2,)),
            pltpu.SemaphoreType.DMA((2,)),
            pltpu.SemaphoreType.DMA((2,)),
            pltpu.SemaphoreType.DMA((2,)),
            pltpu.SemaphoreType.REGULAR,
        ],
        compiler_params=pltpu.CompilerParams(
            dimension_semantics=("arbitrary",),
            vmem_limit_bytes=100 * 1024 * 1024,
        ),
    )(p)


# device time: 1578448 ns/iter; 3.0525x vs baseline; 1.0369x over previous
import jax
import jax.numpy as jnp
from jax import lax
from jax.experimental import pallas as pl
from jax.experimental.pallas import tpu as pltpu

N_DEV = 4
M, N = 8192, 4096
TILE_M = 512
N_TILES = M // TILE_M
CHUNK_N = N // N_DEV
HALF = CHUNK_N // 2
N_HOPS = 2 * (N_DEV - 1)


def kernel(x, w_mat):
    p = jnp.dot(x, w_mat, preferred_element_type=jnp.float32)

    def body(p_ref, out_ref, comm_r, comm_l, send_r, recv_r, send_l, recv_l,
             bar_sem):
        my = lax.axis_index("i")
        left = (my - 1) % N_DEV
        right = (my + 1) % N_DEV

        def hop2(ks_r, kr_r, ks_l, kr_l, slot, accumulate):
            rdma_r = pltpu.make_async_remote_copy(
                src_ref=out_ref.at[:, pl.ds(ks_r * CHUNK_N, HALF)],
                dst_ref=comm_r.at[slot],
                send_sem=send_r.at[slot],
                recv_sem=recv_r.at[slot],
                device_id=(right,),
                device_id_type=pl.DeviceIdType.MESH,
            )
            rdma_l = pltpu.make_async_remote_copy(
                src_ref=out_ref.at[:, pl.ds(ks_l * CHUNK_N + HALF, HALF)],
                dst_ref=comm_l.at[slot],
                send_sem=send_l.at[slot],
                recv_sem=recv_l.at[slot],
                device_id=(left,),
                device_id_type=pl.DeviceIdType.MESH,
            )
            rdma_r.start()
            rdma_l.start()
            rdma_r.wait_recv()
            rdma_l.wait_recv()
            dst_r = out_ref.at[:, pl.ds(kr_r * CHUNK_N, HALF)]
            dst_l = out_ref.at[:, pl.ds(kr_l * CHUNK_N + HALF, HALF)]
            if accumulate:
                dst_r[:, :] = dst_r[:, :] + comm_r[slot, :, :]
                dst_l[:, :] = dst_l[:, :] + comm_l[slot, :, :]
            else:
                dst_r[:, :] = comm_r[slot, :, :]
                dst_l[:, :] = comm_l[slot, :, :]
            rdma_r.wait_send()
            rdma_l.wait_send()

        out_ref[:, :] = p_ref[:, :]

        for s in range(N_DEV - 1):
            hop2(
                (my - s) % N_DEV, (my - s - 1) % N_DEV,
                (my + s) % N_DEV, (my + s + 1) % N_DEV,
                s, True,
            )

        ko_r = (my + 1) % N_DEV
        ko_l = (my - 1) % N_DEV
        own_r = out_ref.at[:, pl.ds(ko_r * CHUNK_N, HALF)]
        own_l = out_ref.at[:, pl.ds(ko_l * CHUNK_N + HALF, HALF)]
        y = own_r[:, :]
        own_r[:, :] = y / (1.0 + jnp.exp(-y))
        y = own_l[:, :]
        own_l[:, :] = y / (1.0 + jnp.exp(-y))

        for s in range(N_DEV - 1):
            hop2(
                (my + 1 - s) % N_DEV, (my - s) % N_DEV,
                (my - 1 + s) % N_DEV, (my + s) % N_DEV,
                N_DEV - 1 + s, False,
            )

        for nbr in (left, right):
            pl.semaphore_signal(
                bar_sem,
                inc=1,
                device_id=(nbr,),
                device_id_type=pl.DeviceIdType.MESH,
            )
        pl.semaphore_wait(bar_sem, 2)

    return pl.pallas_call(
        body,
        grid=(N_TILES,),
        out_shape=jax.ShapeDtypeStruct((M, N), jnp.float32),
        in_specs=[pl.BlockSpec((TILE_M, N), lambda i: (i, 0))],
        out_specs=pl.BlockSpec((TILE_M, N), lambda i: (i, 0)),
        scratch_shapes=[
            pltpu.VMEM((N_HOPS, TILE_M, HALF), jnp.float32),
            pltpu.VMEM((N_HOPS, TILE_M, HALF), jnp.float32),
            pltpu.SemaphoreType.DMA((N_HOPS,)),
            pltpu.SemaphoreType.DMA((N_HOPS,)),
            pltpu.SemaphoreType.DMA((N_HOPS,)),
            pltpu.SemaphoreType.DMA((N_HOPS,)),
            pltpu.SemaphoreType.REGULAR,
        ],
        compiler_params=pltpu.CompilerParams(
            dimension_semantics=("arbitrary",),
            vmem_limit_bytes=100 * 1024 * 1024,
        ),
    )(p)


# device time: 1569408 ns/iter; 3.0700x vs baseline; 1.0058x over previous
import jax
import jax.numpy as jnp
from jax import lax
from jax.experimental import pallas as pl
from jax.experimental.pallas import tpu as pltpu

N_DEV = 4
M, N = 8192, 4096
TILE_M = 512
N_TILES = M // TILE_M
CHUNK_N = N // N_DEV
HALF = CHUNK_N // 2
N_HOPS = 2 * (N_DEV - 1)


def kernel(x, w_mat):
    p = jnp.dot(x, w_mat, preferred_element_type=jnp.float32)

    def body(p_ref, out_ref, comm_r, comm_l, send_r, recv_r, send_l, recv_l,
             bar_sem):
        my = lax.axis_index("i")
        left = (my - 1) % N_DEV
        right = (my + 1) % N_DEV

        def hop2(buf, ks_r, kr_r, ks_l, kr_l, slot, accumulate):
            rdma_r = pltpu.make_async_remote_copy(
                src_ref=buf.at[:, pl.ds(ks_r * CHUNK_N, HALF)],
                dst_ref=comm_r.at[slot],
                send_sem=send_r.at[slot],
                recv_sem=recv_r.at[slot],
                device_id=(right,),
                device_id_type=pl.DeviceIdType.MESH,
            )
            rdma_l = pltpu.make_async_remote_copy(
                src_ref=buf.at[:, pl.ds(ks_l * CHUNK_N + HALF, HALF)],
                dst_ref=comm_l.at[slot],
                send_sem=send_l.at[slot],
                recv_sem=recv_l.at[slot],
                device_id=(left,),
                device_id_type=pl.DeviceIdType.MESH,
            )
            rdma_r.start()
            rdma_l.start()
            rdma_r.wait_recv()
            rdma_l.wait_recv()
            dst_r = buf.at[:, pl.ds(kr_r * CHUNK_N, HALF)]
            dst_l = buf.at[:, pl.ds(kr_l * CHUNK_N + HALF, HALF)]
            if accumulate:
                dst_r[:, :] = dst_r[:, :] + comm_r[slot, :, :]
                dst_l[:, :] = dst_l[:, :] + comm_l[slot, :, :]
            else:
                dst_r[:, :] = comm_r[slot, :, :]
                dst_l[:, :] = comm_l[slot, :, :]
            rdma_r.wait_send()
            rdma_l.wait_send()

        for s in range(N_DEV - 1):
            hop2(
                p_ref,
                (my - s) % N_DEV, (my - s - 1) % N_DEV,
                (my + s) % N_DEV, (my + s + 1) % N_DEV,
                s, True,
            )

        ko_r = (my + 1) % N_DEV
        ko_l = (my - 1) % N_DEV
        y = p_ref[:, pl.ds(ko_r * CHUNK_N, HALF)]
        out_ref[:, pl.ds(ko_r * CHUNK_N, HALF)] = y / (1.0 + jnp.exp(-y))
        y = p_ref[:, pl.ds(ko_l * CHUNK_N + HALF, HALF)]
        out_ref[:, pl.ds(ko_l * CHUNK_N + HALF, HALF)] = y / (1.0 + jnp.exp(-y))

        for s in range(N_DEV - 1):
            hop2(
                out_ref,
                (my + 1 - s) % N_DEV, (my - s) % N_DEV,
                (my - 1 + s) % N_DEV, (my + s) % N_DEV,
                N_DEV - 1 + s, False,
            )

        for nbr in (left, right):
            pl.semaphore_signal(
                bar_sem,
                inc=1,
                device_id=(nbr,),
                device_id_type=pl.DeviceIdType.MESH,
            )
        pl.semaphore_wait(bar_sem, 2)

    return pl.pallas_call(
        body,
        grid=(N_TILES,),
        out_shape=jax.ShapeDtypeStruct((M, N), jnp.float32),
        in_specs=[pl.BlockSpec((TILE_M, N), lambda i: (i, 0))],
        out_specs=pl.BlockSpec((TILE_M, N), lambda i: (i, 0)),
        scratch_shapes=[
            pltpu.VMEM((N_HOPS, TILE_M, HALF), jnp.float32),
            pltpu.VMEM((N_HOPS, TILE_M, HALF), jnp.float32),
            pltpu.SemaphoreType.DMA((N_HOPS,)),
            pltpu.SemaphoreType.DMA((N_HOPS,)),
            pltpu.SemaphoreType.DMA((N_HOPS,)),
            pltpu.SemaphoreType.DMA((N_HOPS,)),
            pltpu.SemaphoreType.REGULAR,
        ],
        compiler_params=pltpu.CompilerParams(
            dimension_semantics=("arbitrary",),
            vmem_limit_bytes=100 * 1024 * 1024,
        ),
    )(p)


# device time: 1405989 ns/iter; 3.4269x vs baseline; 1.1162x over previous
import jax
import jax.numpy as jnp
from jax import lax
from jax.experimental import pallas as pl
from jax.experimental.pallas import tpu as pltpu

N_DEV = 4
M, K, N = 8192, 2048, 4096
TILE_M = 512
N_TILES = M // TILE_M
CHUNK_N = N // N_DEV
HALF = CHUNK_N // 2
N_HOPS = 2 * (N_DEV - 1)


def kernel(x, w_mat):
    def body(x_hbm, w_hbm, out_hbm, w_vmem, x_buf, work,
             comm_r, comm_l, send_r, recv_r, send_l, recv_l,
             w_sem, x_sems, out_sems, bar_sem):
        t = pl.program_id(0)
        my = lax.axis_index("i")
        left = (my - 1) % N_DEV
        right = (my + 1) % N_DEV

        rows = pl.ds(t * TILE_M, TILE_M)

        def rcols(k):
            return pl.ds(k * CHUNK_N, HALF)

        def lcols(k):
            return pl.ds(k * CHUNK_N + HALF, HALF)

        def gemm_half(k, right_half):
            col = k * CHUNK_N + (0 if right_half else HALF)
            xv = x_buf[(t + 1) % 2]
            wv = w_vmem[:, pl.ds(col, HALF)]
            work[:, pl.ds(col, HALF)] = jnp.dot(
                xv, wv, preferred_element_type=jnp.float32
            )

        def gemm_chunk(k):
            col = k * CHUNK_N
            xv = x_buf[(t + 1) % 2]
            wv = w_vmem[:, pl.ds(col, CHUNK_N)]
            work[:, pl.ds(col, CHUNK_N)] = jnp.dot(
                xv, wv, preferred_element_type=jnp.float32
            )

        @pl.when(t == 0)
        def _():
            wcp = pltpu.make_async_copy(w_hbm, w_vmem, w_sem)
            wcp.start()
            xcp = pltpu.make_async_copy(
                x_hbm.at[pl.ds(0, TILE_M), :], x_buf.at[0], x_sems.at[0]
            )
            xcp.start()
            wcp.wait()
            xcp.wait()
            for k in range(N_DEV):
                col = k * CHUNK_N
                work[:, pl.ds(col, CHUNK_N)] = jnp.dot(
                    x_buf[0],
                    w_vmem[:, pl.ds(col, CHUNK_N)],
                    preferred_element_type=jnp.float32,
                )

        @pl.when(t + 1 < N_TILES)
        def _():
            xcp = pltpu.make_async_copy(
                x_hbm.at[pl.ds((t + 1) * TILE_M, TILE_M), :],
                x_buf.at[(t + 1) % 2],
                x_sems.at[(t + 1) % 2],
            )
            xcp.start()

        def make_hop(src_r, dst_r, src_l, dst_l, slot):
            rdma_r = pltpu.make_async_remote_copy(
                src_ref=src_r,
                dst_ref=dst_r,
                send_sem=send_r.at[slot],
                recv_sem=recv_r.at[slot],
                device_id=(right,),
                device_id_type=pl.DeviceIdType.MESH,
            )
            rdma_l = pltpu.make_async_remote_copy(
                src_ref=src_l,
                dst_ref=dst_l,
                send_sem=send_l.at[slot],
                recv_sem=recv_l.at[slot],
                device_id=(left,),
                device_id_type=pl.DeviceIdType.MESH,
            )
            return rdma_r, rdma_l

        have_next = t + 1 < N_TILES

        def maybe_gemm(pieces):
            @pl.when(have_next)
            def _():
                for k, rh in pieces:
                    gemm_half(k, rh)

        for s in range(N_DEV - 1):
            ks_r, kr_r = (my - s) % N_DEV, (my - s - 1) % N_DEV
            ks_l, kr_l = (my + s) % N_DEV, (my + s + 1) % N_DEV
            rdma_r, rdma_l = make_hop(
                work.at[:, rcols(ks_r)], comm_r.at[s],
                work.at[:, lcols(ks_l)], comm_l.at[s],
                s,
            )
            rdma_r.start()
            rdma_l.start()
            if s == 1:
                @pl.when(have_next)
                def _():
                    pltpu.make_async_copy(
                        x_hbm.at[pl.ds((t + 1) * TILE_M, TILE_M), :],
                        x_buf.at[(t + 1) % 2],
                        x_sems.at[(t + 1) % 2],
                    ).wait()
                    gemm_half((my - 0) % N_DEV, True)
                    gemm_half((my - 0) % N_DEV, False)
            elif s == 2:
                maybe_gemm([((my - 1) % N_DEV, True), ((my + 1) % N_DEV, False)])
            rdma_r.wait_recv()
            rdma_l.wait_recv()
            work[:, rcols(kr_r)] = work[:, rcols(kr_r)] + comm_r[s, :, :]
            work[:, lcols(kr_l)] = work[:, lcols(kr_l)] + comm_l[s, :, :]
            rdma_r.wait_send()
            rdma_l.wait_send()

        ko_r = (my + 1) % N_DEV
        ko_l = (my - 1) % N_DEV
        y = work[:, rcols(ko_r)]
        work[:, rcols(ko_r)] = y / (1.0 + jnp.exp(-y))
        y = work[:, lcols(ko_l)]
        work[:, lcols(ko_l)] = y / (1.0 + jnp.exp(-y))

        own_dma_r = pltpu.make_async_copy(
            work.at[:, rcols(ko_r)],
            out_hbm.at[rows, rcols(ko_r)],
            out_sems.at[6],
        )
        own_dma_l = pltpu.make_async_copy(
            work.at[:, lcols(ko_l)],
            out_hbm.at[rows, lcols(ko_l)],
            out_sems.at[7],
        )

        out_dmas = []
        for s in range(N_DEV - 1):
            slot = N_DEV - 1 + s
            kr_r = (my - s) % N_DEV
            kr_l = (my + s) % N_DEV
            if s == 0:
                src_r = work.at[:, rcols(ko_r)]
                src_l = work.at[:, lcols(ko_l)]
            else:
                src_r = comm_r.at[slot - 1]
                src_l = comm_l.at[slot - 1]
            rdma_r, rdma_l = make_hop(
                src_r, comm_r.at[slot], src_l, comm_l.at[slot], slot
            )
            rdma_r.start()
            rdma_l.start()
            if s == 0:
                own_dma_r.start()
                own_dma_l.start()
                maybe_gemm([((my - 2) % N_DEV, True), ((my + 2) % N_DEV, False)])
            elif s == 1:
                maybe_gemm([(ko_r, True), (ko_l, False)])
            rdma_r.wait_recv()
            rdma_l.wait_recv()
            dma_r = pltpu.make_async_copy(
                comm_r.at[slot],
                out_hbm.at[rows, rcols(kr_r)],
                out_sems.at[2 * s],
            )
            dma_l = pltpu.make_async_copy(
                comm_l.at[slot],
                out_hbm.at[rows, lcols(kr_l)],
                out_sems.at[2 * s + 1],
            )
            dma_r.start()
            dma_l.start()
            out_dmas.append((dma_r, dma_l))
            rdma_r.wait_send()
            rdma_l.wait_send()
            if s == 0:
                own_dma_r.wait()
                own_dma_l.wait()

        for dma_r, dma_l in out_dmas:
            dma_r.wait()
            dma_l.wait()
        for nbr in (left, right):
            pl.semaphore_signal(
                bar_sem,
                inc=1,
                device_id=(nbr,),
                device_id_type=pl.DeviceIdType.MESH,
            )
        pl.semaphore_wait(bar_sem, 2)

    return pl.pallas_call(
        body,
        grid=(N_TILES,),
        out_shape=jax.ShapeDtypeStruct((M, N), jnp.float32),
        in_specs=[
            pl.BlockSpec(memory_space=pl.ANY),
            pl.BlockSpec(memory_space=pl.ANY),
        ],
        out_specs=pl.BlockSpec(memory_space=pl.ANY),
        scratch_shapes=[
            pltpu.VMEM((K, N), jnp.float32),
            pltpu.VMEM((2, TILE_M, K), jnp.float32),
            pltpu.VMEM((TILE_M, N), jnp.float32),
            pltpu.VMEM((N_HOPS, TILE_M, HALF), jnp.float32),
            pltpu.VMEM((N_HOPS, TILE_M, HALF), jnp.float32),
            pltpu.SemaphoreType.DMA((N_HOPS,)),
            pltpu.SemaphoreType.DMA((N_HOPS,)),
            pltpu.SemaphoreType.DMA((N_HOPS,)),
            pltpu.SemaphoreType.DMA((N_HOPS,)),
            pltpu.SemaphoreType.DMA,
            pltpu.SemaphoreType.DMA((2,)),
            pltpu.SemaphoreType.DMA((8,)),
            pltpu.SemaphoreType.REGULAR,
        ],
        compiler_params=pltpu.CompilerParams(
            dimension_semantics=("arbitrary",),
            vmem_limit_bytes=100 * 1024 * 1024,
        ),
    )(x, w_mat)


# device time: 1250983 ns/iter; 3.8515x vs baseline; 1.1239x over previous
import jax
import jax.numpy as jnp
from jax import lax
from jax.experimental import pallas as pl
from jax.experimental.pallas import tpu as pltpu

N_DEV = 4
M, K, N = 8192, 2048, 4096
TILE_M = 512
N_TILES = M // TILE_M
CHUNK_N = N // N_DEV
HALF = CHUNK_N // 2
SUB = TILE_M // 2
N_HOPS = 2 * (N_DEV - 1)


def kernel(x, w_mat):
    def body(x_hbm, w_hbm, out_hbm, w_vmem, x_buf, work,
             comm_r, comm_l, send_r, recv_r, send_l, recv_l,
             w_sem, x_sems, out_sems, bar_sem):
        t = pl.program_id(0)
        my = lax.axis_index("i")
        left = (my - 1) % N_DEV
        right = (my + 1) % N_DEV

        rows = pl.ds(t * TILE_M, TILE_M)
        have_next = t + 1 < N_TILES

        def rcols(k):
            return pl.ds(k * CHUNK_N, HALF)

        def lcols(k):
            return pl.ds(k * CHUNK_N + HALF, HALF)

        def subrows(u):
            return pl.ds(u * SUB, SUB)

        def gemm_half(k, right_half):
            col = k * CHUNK_N + (0 if right_half else HALF)
            work[:, pl.ds(col, HALF)] = jnp.dot(
                x_buf[(t + 1) % 2],
                w_vmem[:, pl.ds(col, HALF)],
                preferred_element_type=jnp.float32,
            )

        def maybe_gemm(pieces):
            @pl.when(have_next)
            def _():
                for k, rh in pieces:
                    gemm_half(k, rh)

        @pl.when(t == 0)
        def _():
            wcp = pltpu.make_async_copy(w_hbm, w_vmem, w_sem)
            wcp.start()
            xcp = pltpu.make_async_copy(
                x_hbm.at[pl.ds(0, TILE_M), :], x_buf.at[0], x_sems.at[0]
            )
            xcp.start()
            wcp.wait()
            xcp.wait()
            for k in range(N_DEV):
                col = k * CHUNK_N
                work[:, pl.ds(col, CHUNK_N)] = jnp.dot(
                    x_buf[0],
                    w_vmem[:, pl.ds(col, CHUNK_N)],
                    preferred_element_type=jnp.float32,
                )

        @pl.when(have_next)
        def _():
            pltpu.make_async_copy(
                x_hbm.at[pl.ds((t + 1) * TILE_M, TILE_M), :],
                x_buf.at[(t + 1) % 2],
                x_sems.at[(t + 1) % 2],
            ).start()

        def rdma(src, dst, sems_pair, slot, u, dev):
            send_s, recv_s = sems_pair
            return pltpu.make_async_remote_copy(
                src_ref=src,
                dst_ref=dst,
                send_sem=send_s.at[slot * 2 + u],
                recv_sem=recv_s.at[slot * 2 + u],
                device_id=(dev,),
                device_id_type=pl.DeviceIdType.MESH,
            )

        def rs_rdma(s, u):
            ks_r = (my - s) % N_DEV
            ks_l = (my + s) % N_DEV
            rr = rdma(
                work.at[subrows(u), rcols(ks_r)],
                comm_r.at[s, subrows(u)],
                (send_r, recv_r), s, u, right,
            )
            rl = rdma(
                work.at[subrows(u), lcols(ks_l)],
                comm_l.at[s, subrows(u)],
                (send_l, recv_l), s, u, left,
            )
            return rr, rl

        def rs_add(s, u):
            kr_r = (my - s - 1) % N_DEV
            kr_l = (my + s + 1) % N_DEV
            su = subrows(u)
            work[su, rcols(kr_r)] = work[su, rcols(kr_r)] + comm_r[s, su, :]
            work[su, lcols(kr_l)] = work[su, lcols(kr_l)] + comm_l[s, su, :]

        ko_r = (my + 1) % N_DEV
        ko_l = (my - 1) % N_DEV

        def ag_rdma(s, u):
            slot = N_DEV - 1 + s
            if s == 0:
                src_r = work.at[subrows(u), rcols(ko_r)]
                src_l = work.at[subrows(u), lcols(ko_l)]
            else:
                src_r = comm_r.at[slot - 1, subrows(u)]
                src_l = comm_l.at[slot - 1, subrows(u)]
            rr = rdma(
                src_r, comm_r.at[slot, subrows(u)],
                (send_r, recv_r), slot, u, right,
            )
            rl = rdma(
                src_l, comm_l.at[slot, subrows(u)],
                (send_l, recv_l), slot, u, left,
            )
            return rr, rl

        def silu_own(u):
            su = subrows(u)
            y = work[su, rcols(ko_r)]
            work[su, rcols(ko_r)] = y / (1.0 + jnp.exp(-y))
            y = work[su, lcols(ko_l)]
            work[su, lcols(ko_l)] = y / (1.0 + jnp.exp(-y))

        def wait_x():
            @pl.when(have_next)
            def _():
                pltpu.make_async_copy(
                    x_hbm.at[pl.ds((t + 1) * TILE_M, TILE_M), :],
                    x_buf.at[(t + 1) % 2],
                    x_sems.at[(t + 1) % 2],
                ).wait()

        rs = {(s, u): rs_rdma(s, u) for s in range(3) for u in range(2)}
        ag = {(s, u): ag_rdma(s, u) for s in range(3) for u in range(2)}

        def start(d):
            d[0].start()
            d[1].start()

        def wait_recv(d):
            d[0].wait_recv()
            d[1].wait_recv()

        def wait_send(d):
            d[0].wait_send()
            d[1].wait_send()

        start(rs[0, 0])
        start(rs[0, 1])
        wait_recv(rs[0, 0]); rs_add(0, 0)
        start(rs[1, 0])
        wait_recv(rs[0, 1]); rs_add(0, 1)
        start(rs[1, 1])
        wait_send(rs[0, 0]); wait_send(rs[0, 1])
        wait_x()
        maybe_gemm([(my % N_DEV, True), (my % N_DEV, False)])
        wait_recv(rs[1, 0]); rs_add(1, 0)
        start(rs[2, 0])
        wait_recv(rs[1, 1]); rs_add(1, 1)
        start(rs[2, 1])
        wait_send(rs[1, 0]); wait_send(rs[1, 1])
        maybe_gemm([((my - 1) % N_DEV, True), ((my + 1) % N_DEV, False)])
        wait_recv(rs[2, 0]); rs_add(2, 0); silu_own(0)
        start(ag[0, 0])
        wait_recv(rs[2, 1]); rs_add(2, 1); silu_own(1)
        start(ag[0, 1])
        wait_send(rs[2, 0]); wait_send(rs[2, 1])
        maybe_gemm([((my - 2) % N_DEV, True), ((my + 2) % N_DEV, False)])

        own_dma_r = pltpu.make_async_copy(
            work.at[:, rcols(ko_r)], out_hbm.at[rows, rcols(ko_r)],
            out_sems.at[6],
        )
        own_dma_l = pltpu.make_async_copy(
            work.at[:, lcols(ko_l)], out_hbm.at[rows, lcols(ko_l)],
            out_sems.at[7],
        )
        own_dma_r.start()
        own_dma_l.start()

        def out_dma(s):
            slot = N_DEV - 1 + s
            kr_r = (my - s) % N_DEV
            kr_l = (my + s) % N_DEV
            dr = pltpu.make_async_copy(
                comm_r.at[slot], out_hbm.at[rows, rcols(kr_r)],
                out_sems.at[2 * s],
            )
            dl = pltpu.make_async_copy(
                comm_l.at[slot], out_hbm.at[rows, lcols(kr_l)],
                out_sems.at[2 * s + 1],
            )
            dr.start()
            dl.start()
            return dr, dl

        wait_recv(ag[0, 0])
        start(ag[1, 0])
        wait_recv(ag[0, 1])
        start(ag[1, 1])
        dmas0 = out_dma(0)
        wait_send(ag[0, 0]); wait_send(ag[0, 1])
        own_dma_r.wait()
        own_dma_l.wait()
        maybe_gemm([(ko_r, True), (ko_l, False)])
        wait_recv(ag[1, 0])
        start(ag[2, 0])
        wait_recv(ag[1, 1])
        start(ag[2, 1])
        dmas1 = out_dma(1)
        wait_send(ag[1, 0]); wait_send(ag[1, 1])
        wait_recv(ag[2, 0])
        wait_recv(ag[2, 1])
        dmas2 = out_dma(2)
        wait_send(ag[2, 0]); wait_send(ag[2, 1])

        for dr, dl in (dmas0, dmas1, dmas2):
            dr.wait()
            dl.wait()

        for nbr in (left, right):
            pl.semaphore_signal(
                bar_sem,
                inc=1,
                device_id=(nbr,),
                device_id_type=pl.DeviceIdType.MESH,
            )
        pl.semaphore_wait(bar_sem, 2)

    return pl.pallas_call(
        body,
        grid=(N_TILES,),
        out_shape=jax.ShapeDtypeStruct((M, N), jnp.float32),
        in_specs=[
            pl.BlockSpec(memory_space=pl.ANY),
            pl.BlockSpec(memory_space=pl.ANY),
        ],
        out_specs=pl.BlockSpec(memory_space=pl.ANY),
        scratch_shapes=[
            pltpu.VMEM((K, N), jnp.float32),
            pltpu.VMEM((2, TILE_M, K), jnp.float32),
            pltpu.VMEM((TILE_M, N), jnp.float32),
            pltpu.VMEM((N_HOPS, TILE_M, HALF), jnp.float32),
            pltpu.VMEM((N_HOPS, TILE_M, HALF), jnp.float32),
            pltpu.SemaphoreType.DMA((N_HOPS * 2,)),
            pltpu.SemaphoreType.DMA((N_HOPS * 2,)),
            pltpu.SemaphoreType.DMA((N_HOPS * 2,)),
            pltpu.SemaphoreType.DMA((N_HOPS * 2,)),
            pltpu.SemaphoreType.DMA,
            pltpu.SemaphoreType.DMA((2,)),
            pltpu.SemaphoreType.DMA((8,)),
            pltpu.SemaphoreType.REGULAR,
        ],
        compiler_params=pltpu.CompilerParams(
            dimension_semantics=("arbitrary",),
            vmem_limit_bytes=100 * 1024 * 1024,
        ),
    )(x, w_mat)


# device time: 1239287 ns/iter; 3.8878x vs baseline; 1.0094x over previous
import jax
import jax.numpy as jnp
from jax import lax
from jax.experimental import pallas as pl
from jax.experimental.pallas import tpu as pltpu

N_DEV = 4
M, K, N = 8192, 2048, 4096
TILE_M = 512
N_TILES = M // TILE_M
CHUNK_N = N // N_DEV
HALF = CHUNK_N // 2
SUB = TILE_M // 2
N_HOPS = 2 * (N_DEV - 1)


def kernel(x, w_mat):
    def body(x_hbm, w_hbm, out_hbm, w_vmem, x_buf, work,
             comm_r, comm_l, send_r, recv_r, send_l, recv_l,
             w_sem, x_sems, out_sems):
        t = pl.program_id(0)
        my = lax.axis_index("i")
        left = (my - 1) % N_DEV
        right = (my + 1) % N_DEV

        rows = pl.ds(t * TILE_M, TILE_M)
        have_next = t + 1 < N_TILES

        def rcols(k):
            return pl.ds(k * CHUNK_N, HALF)

        def lcols(k):
            return pl.ds(k * CHUNK_N + HALF, HALF)

        def subrows(u):
            return pl.ds(u * SUB, SUB)

        def gemm_half(k, right_half):
            col = k * CHUNK_N + (0 if right_half else HALF)
            work[:, pl.ds(col, HALF)] = jnp.dot(
                x_buf[(t + 1) % 2],
                w_vmem[:, pl.ds(col, HALF)],
                preferred_element_type=jnp.float32,
            )

        def maybe_gemm(pieces):
            @pl.when(have_next)
            def _():
                for k, rh in pieces:
                    gemm_half(k, rh)

        @pl.when(t == 0)
        def _():
            wcp = pltpu.make_async_copy(w_hbm, w_vmem, w_sem)
            wcp.start()
            xcp = pltpu.make_async_copy(
                x_hbm.at[pl.ds(0, TILE_M), :], x_buf.at[0], x_sems.at[0]
            )
            xcp.start()
            wcp.wait()
            xcp.wait()
            for k in range(N_DEV):
                col = k * CHUNK_N
                work[:, pl.ds(col, CHUNK_N)] = jnp.dot(
                    x_buf[0],
                    w_vmem[:, pl.ds(col, CHUNK_N)],
                    preferred_element_type=jnp.float32,
                )

        @pl.when(have_next)
        def _():
            pltpu.make_async_copy(
                x_hbm.at[pl.ds((t + 1) * TILE_M, TILE_M), :],
                x_buf.at[(t + 1) % 2],
                x_sems.at[(t + 1) % 2],
            ).start()

        def rdma(src, dst, sems_pair, slot, u, dev):
            send_s, recv_s = sems_pair
            return pltpu.make_async_remote_copy(
                src_ref=src,
                dst_ref=dst,
                send_sem=send_s.at[slot * 2 + u],
                recv_sem=recv_s.at[slot * 2 + u],
                device_id=(dev,),
                device_id_type=pl.DeviceIdType.MESH,
            )

        def rs_rdma(s, u):
            ks_r = (my - s) % N_DEV
            ks_l = (my + s) % N_DEV
            rr = rdma(
                work.at[subrows(u), rcols(ks_r)],
                comm_r.at[s, subrows(u)],
                (send_r, recv_r), s, u, right,
            )
            rl = rdma(
                work.at[subrows(u), lcols(ks_l)],
                comm_l.at[s, subrows(u)],
                (send_l, recv_l), s, u, left,
            )
            return rr, rl

        def rs_add(s, u):
            kr_r = (my - s - 1) % N_DEV
            kr_l = (my + s + 1) % N_DEV
            su = subrows(u)
            work[su, rcols(kr_r)] = work[su, rcols(kr_r)] + comm_r[s, su, :]
            work[su, lcols(kr_l)] = work[su, lcols(kr_l)] + comm_l[s, su, :]

        ko_r = (my + 1) % N_DEV
        ko_l = (my - 1) % N_DEV

        def ag_rdma(s, u):
            slot = N_DEV - 1 + s
            if s == 0:
                src_r = work.at[subrows(u), rcols(ko_r)]
                src_l = work.at[subrows(u), lcols(ko_l)]
            else:
                src_r = comm_r.at[slot - 1, subrows(u)]
                src_l = comm_l.at[slot - 1, subrows(u)]
            rr = rdma(
                src_r, comm_r.at[slot, subrows(u)],
                (send_r, recv_r), slot, u, right,
            )
            rl = rdma(
                src_l, comm_l.at[slot, subrows(u)],
                (send_l, recv_l), slot, u, left,
            )
            return rr, rl

        def silu_own(u):
            su = subrows(u)
            y = work[su, rcols(ko_r)]
            work[su, rcols(ko_r)] = y / (1.0 + jnp.exp(-y))
            y = work[su, lcols(ko_l)]
            work[su, lcols(ko_l)] = y / (1.0 + jnp.exp(-y))

        def wait_x():
            @pl.when(have_next)
            def _():
                pltpu.make_async_copy(
                    x_hbm.at[pl.ds((t + 1) * TILE_M, TILE_M), :],
                    x_buf.at[(t + 1) % 2],
                    x_sems.at[(t + 1) % 2],
                ).wait()

        rs = {(s, u): rs_rdma(s, u) for s in range(3) for u in range(2)}
        ag = {(s, u): ag_rdma(s, u) for s in range(3) for u in range(2)}

        def start(d):
            d[0].start()
            d[1].start()

        def wait_recv(d):
            d[0].wait_recv()
            d[1].wait_recv()

        def wait_send(d):
            d[0].wait_send()
            d[1].wait_send()

        start(rs[0, 0])
        start(rs[0, 1])
        wait_recv(rs[0, 0]); rs_add(0, 0)
        start(rs[1, 0])
        wait_recv(rs[0, 1]); rs_add(0, 1)
        start(rs[1, 1])
        wait_send(rs[0, 0]); wait_send(rs[0, 1])
        wait_x()
        maybe_gemm([(my % N_DEV, True), (my % N_DEV, False)])
        wait_recv(rs[1, 0]); rs_add(1, 0)
        start(rs[2, 0])
        wait_recv(rs[1, 1]); rs_add(1, 1)
        start(rs[2, 1])
        wait_send(rs[1, 0]); wait_send(rs[1, 1])
        maybe_gemm([((my - 1) % N_DEV, True), ((my + 1) % N_DEV, False)])
        wait_recv(rs[2, 0]); rs_add(2, 0); silu_own(0)
        start(ag[0, 0])
        wait_recv(rs[2, 1]); rs_add(2, 1); silu_own(1)
        start(ag[0, 1])
        wait_send(rs[2, 0]); wait_send(rs[2, 1])
        maybe_gemm([((my - 2) % N_DEV, True), ((my + 2) % N_DEV, False)])

        own_dma_r = pltpu.make_async_copy(
            work.at[:, rcols(ko_r)], out_hbm.at[rows, rcols(ko_r)],
            out_sems.at[6],
        )
        own_dma_l = pltpu.make_async_copy(
            work.at[:, lcols(ko_l)], out_hbm.at[rows, lcols(ko_l)],
            out_sems.at[7],
        )
        own_dma_r.start()
        own_dma_l.start()

        def out_dma(s):
            slot = N_DEV - 1 + s
            kr_r = (my - s) % N_DEV
            kr_l = (my + s) % N_DEV
            dr = pltpu.make_async_copy(
                comm_r.at[slot], out_hbm.at[rows, rcols(kr_r)],
                out_sems.at[2 * s],
            )
            dl = pltpu.make_async_copy(
                comm_l.at[slot], out_hbm.at[rows, lcols(kr_l)],
                out_sems.at[2 * s + 1],
            )
            dr.start()
            dl.start()
            return dr, dl

        wait_recv(ag[0, 0])
        start(ag[1, 0])
        wait_recv(ag[0, 1])
        start(ag[1, 1])
        dmas0 = out_dma(0)
        wait_send(ag[0, 0]); wait_send(ag[0, 1])
        own_dma_r.wait()
        own_dma_l.wait()
        maybe_gemm([(ko_r, True), (ko_l, False)])
        wait_recv(ag[1, 0])
        start(ag[2, 0])
        wait_recv(ag[1, 1])
        start(ag[2, 1])
        dmas1 = out_dma(1)
        wait_send(ag[1, 0]); wait_send(ag[1, 1])
        wait_recv(ag[2, 0])
        wait_recv(ag[2, 1])
        dmas2 = out_dma(2)
        wait_send(ag[2, 0]); wait_send(ag[2, 1])

        for dr, dl in (dmas0, dmas1, dmas2):
            dr.wait()
            dl.wait()


    return pl.pallas_call(
        body,
        grid=(N_TILES,),
        out_shape=jax.ShapeDtypeStruct((M, N), jnp.float32),
        in_specs=[
            pl.BlockSpec(memory_space=pl.ANY),
            pl.BlockSpec(memory_space=pl.ANY),
        ],
        out_specs=pl.BlockSpec(memory_space=pl.ANY),
        scratch_shapes=[
            pltpu.VMEM((K, N), jnp.float32),
            pltpu.VMEM((2, TILE_M, K), jnp.float32),
            pltpu.VMEM((TILE_M, N), jnp.float32),
            pltpu.VMEM((N_HOPS, TILE_M, HALF), jnp.float32),
            pltpu.VMEM((N_HOPS, TILE_M, HALF), jnp.float32),
            pltpu.SemaphoreType.DMA((N_HOPS * 2,)),
            pltpu.SemaphoreType.DMA((N_HOPS * 2,)),
            pltpu.SemaphoreType.DMA((N_HOPS * 2,)),
            pltpu.SemaphoreType.DMA((N_HOPS * 2,)),
            pltpu.SemaphoreType.DMA,
            pltpu.SemaphoreType.DMA((2,)),
            pltpu.SemaphoreType.DMA((8,)),
        ],
        compiler_params=pltpu.CompilerParams(
            dimension_semantics=("arbitrary",),
            vmem_limit_bytes=100 * 1024 * 1024,
        ),
    )(x, w_mat)


# device time: 1220992 ns/iter; 3.9461x vs baseline; 1.0150x over previous
import jax
import jax.numpy as jnp
from jax import lax
from jax.experimental import pallas as pl
from jax.experimental.pallas import tpu as pltpu

N_DEV = 4
M, K, N = 8192, 2048, 4096
TILE_M = 512
N_TILES = M // TILE_M
CHUNK_N = N // N_DEV
HALF = CHUNK_N // 2
NSUB = 4
SUB = TILE_M // NSUB
N_HOPS = 2 * (N_DEV - 1)


def kernel(x, w_mat):
    def body(x_hbm, w_hbm, out_hbm, w_vmem, x_buf, work,
             comm_r, comm_l, send_r, recv_r, send_l, recv_l,
             w_sem, x_sems, out_sems):
        t = pl.program_id(0)
        my = lax.axis_index("i")
        left = (my - 1) % N_DEV
        right = (my + 1) % N_DEV

        rows = pl.ds(t * TILE_M, TILE_M)
        have_next = t + 1 < N_TILES

        def rcols(k):
            return pl.ds(k * CHUNK_N, HALF)

        def lcols(k):
            return pl.ds(k * CHUNK_N + HALF, HALF)

        def subrows(u):
            return pl.ds(u * SUB, SUB)

        def gemm_half(k, right_half):
            col = k * CHUNK_N + (0 if right_half else HALF)
            work[:, pl.ds(col, HALF)] = jnp.dot(
                x_buf[(t + 1) % 2],
                w_vmem[:, pl.ds(col, HALF)],
                preferred_element_type=jnp.float32,
            )

        def maybe_gemm(pieces):
            @pl.when(have_next)
            def _():
                for k, rh in pieces:
                    gemm_half(k, rh)

        @pl.when(t == 0)
        def _():
            wcp = pltpu.make_async_copy(w_hbm, w_vmem, w_sem)
            wcp.start()
            xcp = pltpu.make_async_copy(
                x_hbm.at[pl.ds(0, TILE_M), :], x_buf.at[0], x_sems.at[0]
            )
            xcp.start()
            wcp.wait()
            xcp.wait()
            for k in range(N_DEV):
                col = k * CHUNK_N
                work[:, pl.ds(col, CHUNK_N)] = jnp.dot(
                    x_buf[0],
                    w_vmem[:, pl.ds(col, CHUNK_N)],
                    preferred_element_type=jnp.float32,
                )

        @pl.when(have_next)
        def _():
            pltpu.make_async_copy(
                x_hbm.at[pl.ds((t + 1) * TILE_M, TILE_M), :],
                x_buf.at[(t + 1) % 2],
                x_sems.at[(t + 1) % 2],
            ).start()

        def wait_x():
            @pl.when(have_next)
            def _():
                pltpu.make_async_copy(
                    x_hbm.at[pl.ds((t + 1) * TILE_M, TILE_M), :],
                    x_buf.at[(t + 1) % 2],
                    x_sems.at[(t + 1) % 2],
                ).wait()

        def rdma(src, dst, sems_pair, slot, u, dev):
            send_s, recv_s = sems_pair
            return pltpu.make_async_remote_copy(
                src_ref=src,
                dst_ref=dst,
                send_sem=send_s.at[slot * NSUB + u],
                recv_sem=recv_s.at[slot * NSUB + u],
                device_id=(dev,),
                device_id_type=pl.DeviceIdType.MESH,
            )

        def rs_rdma(s, u):
            ks_r = (my - s) % N_DEV
            ks_l = (my + s) % N_DEV
            rr = rdma(
                work.at[subrows(u), rcols(ks_r)],
                comm_r.at[s, subrows(u)],
                (send_r, recv_r), s, u, right,
            )
            rl = rdma(
                work.at[subrows(u), lcols(ks_l)],
                comm_l.at[s, subrows(u)],
                (send_l, recv_l), s, u, left,
            )
            return rr, rl

        def rs_add(s, u):
            kr_r = (my - s - 1) % N_DEV
            kr_l = (my + s + 1) % N_DEV
            su = subrows(u)
            work[su, rcols(kr_r)] = work[su, rcols(kr_r)] + comm_r[s, su, :]
            work[su, lcols(kr_l)] = work[su, lcols(kr_l)] + comm_l[s, su, :]

        ko_r = (my + 1) % N_DEV
        ko_l = (my - 1) % N_DEV

        def ag_rdma(s, u):
            slot = N_DEV - 1 + s
            if s == 0:
                src_r = work.at[subrows(u), rcols(ko_r)]
                src_l = work.at[subrows(u), lcols(ko_l)]
            else:
                src_r = comm_r.at[slot - 1, subrows(u)]
                src_l = comm_l.at[slot - 1, subrows(u)]
            rr = rdma(
                src_r, comm_r.at[slot, subrows(u)],
                (send_r, recv_r), slot, u, right,
            )
            rl = rdma(
                src_l, comm_l.at[slot, subrows(u)],
                (send_l, recv_l), slot, u, left,
            )
            return rr, rl

        def silu_own(u):
            su = subrows(u)
            y = work[su, rcols(ko_r)]
            work[su, rcols(ko_r)] = y / (1.0 + jnp.exp(-y))
            y = work[su, lcols(ko_l)]
            work[su, lcols(ko_l)] = y / (1.0 + jnp.exp(-y))

        rs = {(s, u): rs_rdma(s, u) for s in range(3) for u in range(NSUB)}
        ag = {(s, u): ag_rdma(s, u) for s in range(3) for u in range(NSUB)}

        def start(d):
            d[0].start()
            d[1].start()

        def wait_recv(d):
            d[0].wait_recv()
            d[1].wait_recv()

        def wait_send(d):
            d[0].wait_send()
            d[1].wait_send()

        @pl.when(t == 0)
        def _():
            for u in range(NSUB):
                start(rs[0, u])

        gemm_after_rs = {
            0: [(0, True), (0, False)],
            1: [(-1, True), (1, False)],
            2: [(-2, True), (2, False)],
        }
        for s in range(3):
            for u in range(NSUB):
                wait_recv(rs[s, u])
                rs_add(s, u)
                if s < 2:
                    start(rs[s + 1, u])
                else:
                    silu_own(u)
                    start(ag[0, u])
            for u in range(NSUB):
                wait_send(rs[s, u])
            if s == 0:
                wait_x()
            maybe_gemm(
                [((my + d) % N_DEV, rh) for d, rh in gemm_after_rs[s]]
            )

        own_dma_r = pltpu.make_async_copy(
            work.at[:, rcols(ko_r)], out_hbm.at[rows, rcols(ko_r)],
            out_sems.at[6],
        )
        own_dma_l = pltpu.make_async_copy(
            work.at[:, lcols(ko_l)], out_hbm.at[rows, lcols(ko_l)],
            out_sems.at[7],
        )
        own_dma_r.start()
        own_dma_l.start()

        def out_dma(s):
            slot = N_DEV - 1 + s
            kr_r = (my - s) % N_DEV
            kr_l = (my + s) % N_DEV
            dr = pltpu.make_async_copy(
                comm_r.at[slot], out_hbm.at[rows, rcols(kr_r)],
                out_sems.at[2 * s],
            )
            dl = pltpu.make_async_copy(
                comm_l.at[slot], out_hbm.at[rows, lcols(kr_l)],
                out_sems.at[2 * s + 1],
            )
            dr.start()
            dl.start()
            return dr, dl

        dmas = []
        for s in range(3):
            for u in range(NSUB):
                wait_recv(ag[s, u])
                if s < 2:
                    start(ag[s + 1, u])
            dmas.append(out_dma(s))
            for u in range(NSUB):
                wait_send(ag[s, u])
            if s == 0:
                own_dma_r.wait()
                own_dma_l.wait()
                maybe_gemm([(ko_r, True), (ko_l, False)])

        @pl.when(have_next)
        def _():
            for u in range(NSUB):
                start(rs[0, u])

        for dr, dl in dmas:
            dr.wait()
            dl.wait()

    return pl.pallas_call(
        body,
        grid=(N_TILES,),
        out_shape=jax.ShapeDtypeStruct((M, N), jnp.float32),
        in_specs=[
            pl.BlockSpec(memory_space=pl.ANY),
            pl.BlockSpec(memory_space=pl.ANY),
        ],
        out_specs=pl.BlockSpec(memory_space=pl.ANY),
        scratch_shapes=[
            pltpu.VMEM((K, N), jnp.float32),
            pltpu.VMEM((2, TILE_M, K), jnp.float32),
            pltpu.VMEM((TILE_M, N), jnp.float32),
            pltpu.VMEM((N_HOPS, TILE_M, HALF), jnp.float32),
            pltpu.VMEM((N_HOPS, TILE_M, HALF), jnp.float32),
            pltpu.SemaphoreType.DMA((N_HOPS * NSUB,)),
            pltpu.SemaphoreType.DMA((N_HOPS * NSUB,)),
            pltpu.SemaphoreType.DMA((N_HOPS * NSUB,)),
            pltpu.SemaphoreType.DMA((N_HOPS * NSUB,)),
            pltpu.SemaphoreType.DMA,
            pltpu.SemaphoreType.DMA((2,)),
            pltpu.SemaphoreType.DMA((8,)),
        ],
        compiler_params=pltpu.CompilerParams(
            dimension_semantics=("arbitrary",),
            vmem_limit_bytes=100 * 1024 * 1024,
        ),
    )(x, w_mat)


# device time: 1209724 ns/iter; 3.9829x vs baseline; 1.0093x over previous
import jax
import jax.numpy as jnp
from jax import lax
from jax.experimental import pallas as pl
from jax.experimental.pallas import tpu as pltpu

N_DEV = 4
M, K, N = 8192, 2048, 4096
TILE_M = 512
N_TILES = M // TILE_M
CHUNK_N = N // N_DEV
HALF = CHUNK_N // 2
NSUB = 4
SUB = TILE_M // NSUB
N_HOPS = 2 * (N_DEV - 1)


def kernel(x, w_mat):
    def body(x_hbm, w_hbm, out_hbm, w_vmem, x_buf, work,
             comm_r, comm_l, send_r, recv_r, send_l, recv_l,
             w_sem, x_sems, out_sems):
        t = pl.program_id(0)
        my = lax.axis_index("i")
        left = (my - 1) % N_DEV
        right = (my + 1) % N_DEV

        rows = pl.ds(t * TILE_M, TILE_M)
        have_next = t + 1 < N_TILES

        def rcols(k):
            return pl.ds(k * CHUNK_N, HALF)

        def lcols(k):
            return pl.ds(k * CHUNK_N + HALF, HALF)

        def subrows(u):
            return pl.ds(u * SUB, SUB)

        def gemm_half(k, right_half):
            col = k * CHUNK_N + (0 if right_half else HALF)
            work[:, pl.ds(col, HALF)] = jnp.dot(
                x_buf[(t + 1) % 2],
                w_vmem[:, pl.ds(col, HALF)],
                preferred_element_type=jnp.float32,
            )

        def maybe_gemm(pieces):
            @pl.when(have_next)
            def _():
                for k, rh in pieces:
                    gemm_half(k, rh)

        def w_chunk_copy(j):
            col = ((my + j) % N_DEV) * CHUNK_N
            return pltpu.make_async_copy(
                w_hbm.at[:, pl.ds(col, CHUNK_N)],
                w_vmem.at[:, pl.ds(col, CHUNK_N)],
                w_sem.at[j],
            )

        def gemm_tile0(j):
            col = ((my + j) % N_DEV) * CHUNK_N
            work[:, pl.ds(col, CHUNK_N)] = jnp.dot(
                x_buf[0],
                w_vmem[:, pl.ds(col, CHUNK_N)],
                preferred_element_type=jnp.float32,
            )

        @pl.when(t == 0)
        def _():
            xcp = pltpu.make_async_copy(
                x_hbm.at[pl.ds(0, TILE_M), :], x_buf.at[0], x_sems.at[0]
            )
            xcp.start()
            for j in range(N_DEV):
                w_chunk_copy(j).start()
            xcp.wait()
            w_chunk_copy(0).wait()
            gemm_tile0(0)

        @pl.when(have_next)
        def _():
            pltpu.make_async_copy(
                x_hbm.at[pl.ds((t + 1) * TILE_M, TILE_M), :],
                x_buf.at[(t + 1) % 2],
                x_sems.at[(t + 1) % 2],
            ).start()

        def wait_x():
            @pl.when(have_next)
            def _():
                pltpu.make_async_copy(
                    x_hbm.at[pl.ds((t + 1) * TILE_M, TILE_M), :],
                    x_buf.at[(t + 1) % 2],
                    x_sems.at[(t + 1) % 2],
                ).wait()

        def rdma(src, dst, sems_pair, slot, u, dev):
            send_s, recv_s = sems_pair
            return pltpu.make_async_remote_copy(
                src_ref=src,
                dst_ref=dst,
                send_sem=send_s.at[slot * NSUB + u],
                recv_sem=recv_s.at[slot * NSUB + u],
                device_id=(dev,),
                device_id_type=pl.DeviceIdType.MESH,
            )

        def rs_rdma(s, u):
            ks_r = (my - s) % N_DEV
            ks_l = (my + s) % N_DEV
            rr = rdma(
                work.at[subrows(u), rcols(ks_r)],
                comm_r.at[s, subrows(u)],
                (send_r, recv_r), s, u, right,
            )
            rl = rdma(
                work.at[subrows(u), lcols(ks_l)],
                comm_l.at[s, subrows(u)],
                (send_l, recv_l), s, u, left,
            )
            return rr, rl

        def rs_add(s, u):
            kr_r = (my - s - 1) % N_DEV
            kr_l = (my + s + 1) % N_DEV
            su = subrows(u)
            work[su, rcols(kr_r)] = work[su, rcols(kr_r)] + comm_r[s, su, :]
            work[su, lcols(kr_l)] = work[su, lcols(kr_l)] + comm_l[s, su, :]

        ko_r = (my + 1) % N_DEV
        ko_l = (my - 1) % N_DEV

        def ag_rdma(s, u):
            slot = N_DEV - 1 + s
            if s == 0:
                src_r = work.at[subrows(u), rcols(ko_r)]
                src_l = work.at[subrows(u), lcols(ko_l)]
            else:
                src_r = comm_r.at[slot - 1, subrows(u)]
                src_l = comm_l.at[slot - 1, subrows(u)]
            rr = rdma(
                src_r, comm_r.at[slot, subrows(u)],
                (send_r, recv_r), slot, u, right,
            )
            rl = rdma(
                src_l, comm_l.at[slot, subrows(u)],
                (send_l, recv_l), slot, u, left,
            )
            return rr, rl

        def silu_own(u):
            su = subrows(u)
            y = work[su, rcols(ko_r)]
            work[su, rcols(ko_r)] = y / (1.0 + jnp.exp(-y))
            y = work[su, lcols(ko_l)]
            work[su, lcols(ko_l)] = y / (1.0 + jnp.exp(-y))

        rs = {(s, u): rs_rdma(s, u) for s in range(3) for u in range(NSUB)}
        ag = {(s, u): ag_rdma(s, u) for s in range(3) for u in range(NSUB)}

        def start(d):
            d[0].start()
            d[1].start()

        def wait_recv(d):
            d[0].wait_recv()
            d[1].wait_recv()

        def wait_send(d):
            d[0].wait_send()
            d[1].wait_send()

        @pl.when(t == 0)
        def _():
            for u in range(NSUB):
                start(rs[0, u])

        @pl.when(t == 0)
        def _():
            for j in range(1, N_DEV):
                w_chunk_copy(j).wait()
                gemm_tile0(j)

        gemm_after_rs = {
            0: [(0, True), (0, False)],
            1: [(-1, True), (1, False)],
            2: [(-2, True), (2, False)],
        }
        for s in range(3):
            for u in range(NSUB):
                wait_recv(rs[s, u])
                rs_add(s, u)
                if s < 2:
                    start(rs[s + 1, u])
                else:
                    silu_own(u)
                    start(ag[0, u])
            for u in range(NSUB):
                wait_send(rs[s, u])
            if s == 0:
                wait_x()
            maybe_gemm(
                [((my + d) % N_DEV, rh) for d, rh in gemm_after_rs[s]]
            )

        own_dma_r = pltpu.make_async_copy(
            work.at[:, rcols(ko_r)], out_hbm.at[rows, rcols(ko_r)],
            out_sems.at[6],
        )
        own_dma_l = pltpu.make_async_copy(
            work.at[:, lcols(ko_l)], out_hbm.at[rows, lcols(ko_l)],
            out_sems.at[7],
        )
        own_dma_r.start()
        own_dma_l.start()

        def out_dma(s):
            slot = N_DEV - 1 + s
            kr_r = (my - s) % N_DEV
            kr_l = (my + s) % N_DEV
            dr = pltpu.make_async_copy(
                comm_r.at[slot], out_hbm.at[rows, rcols(kr_r)],
                out_sems.at[2 * s],
            )
            dl = pltpu.make_async_copy(
                comm_l.at[slot], out_hbm.at[rows, lcols(kr_l)],
                out_sems.at[2 * s + 1],
            )
            dr.start()
            dl.start()
            return dr, dl

        dmas = []
        for s in range(3):
            for u in range(NSUB):
                wait_recv(ag[s, u])
                if s < 2:
                    start(ag[s + 1, u])
            dmas.append(out_dma(s))
            for u in range(NSUB):
                wait_send(ag[s, u])
            if s == 0:
                own_dma_r.wait()
                own_dma_l.wait()
                maybe_gemm([(ko_r, True), (ko_l, False)])

        @pl.when(have_next)
        def _():
            for u in range(NSUB):
                start(rs[0, u])

        for dr, dl in dmas:
            dr.wait()
            dl.wait()

    return pl.pallas_call(
        body,
        grid=(N_TILES,),
        out_shape=jax.ShapeDtypeStruct((M, N), jnp.float32),
        in_specs=[
            pl.BlockSpec(memory_space=pl.ANY),
            pl.BlockSpec(memory_space=pl.ANY),
        ],
        out_specs=pl.BlockSpec(memory_space=pl.ANY),
        scratch_shapes=[
            pltpu.VMEM((K, N), jnp.float32),
            pltpu.VMEM((2, TILE_M, K), jnp.float32),
            pltpu.VMEM((TILE_M, N), jnp.float32),
            pltpu.VMEM((N_HOPS, TILE_M, HALF), jnp.float32),
            pltpu.VMEM((N_HOPS, TILE_M, HALF), jnp.float32),
            pltpu.SemaphoreType.DMA((N_HOPS * NSUB,)),
            pltpu.SemaphoreType.DMA((N_HOPS * NSUB,)),
            pltpu.SemaphoreType.DMA((N_HOPS * NSUB,)),
            pltpu.SemaphoreType.DMA((N_HOPS * NSUB,)),
            pltpu.SemaphoreType.DMA((N_DEV,)),
            pltpu.SemaphoreType.DMA((2,)),
            pltpu.SemaphoreType.DMA((8,)),
        ],
        compiler_params=pltpu.CompilerParams(
            dimension_semantics=("arbitrary",),
            vmem_limit_bytes=100 * 1024 * 1024,
        ),
    )(x, w_mat)
